# baseline (speedup 1.0000x reference)
import functools

import jax
import jax.numpy as jnp
from jax import lax
from jax.experimental import pallas as pl
from jax.experimental.pallas import tpu as pltpu
from jax.experimental.pallas import tpu_sc as plsc

_H = 64
_NB = 4


@functools.lru_cache(maxsize=None)
def _build_sc_kernel(N, RB):
    NW = 32
    per_w = N // NW
    n_chunks = per_w // RB
    assert per_w % RB == 0 and n_chunks >= _NB

    mesh = plsc.VectorSubcoreMesh(core_axis_name="c", subcore_axis_name="s")

    @functools.partial(
        pl.kernel,
        mesh=mesh,
        out_type=jax.ShapeDtypeStruct((N, _H), jnp.float32),
        scratch_types=[
            pltpu.VMEM((_NB, 3, RB), jnp.int32),
            pltpu.VMEM((_NB, RB, _H), jnp.float32),
            [pltpu.SemaphoreType.DMA] * _NB,
            [pltpu.SemaphoreType.DMA] * _NB,
            [pltpu.SemaphoreType.DMA] * _NB,
            [pltpu.SemaphoreType.DMA] * _NB,
        ],
        compiler_params=pltpu.CompilerParams(use_tc_tiling_on_sc=False),
    )
    def sc_kernel(ids, qtab, ctab, ptab, out, ids_v, buf, si, sc, sg, so):
        wid = lax.axis_index("s") * 2 + lax.axis_index("c")
        chunk0 = wid * n_chunks

        def ids_copy(g, slot):
            return pltpu.make_async_copy(ids.at[chunk0 + g], ids_v.at[slot],
                                         si[slot])

        def c_copy(g, slot):
            return pltpu.make_async_copy(ctab.at[ids_v.at[slot, 1]],
                                         buf.at[slot], sc[slot])

        def out_copy(g, slot):
            off = (chunk0 + g) * RB
            return pltpu.make_async_copy(buf.at[slot],
                                         out.at[pl.ds(off, RB)], so[slot])

        ids_copy(0, 0).start()
        ids_copy(1, 1).start()

        def step_slot(g, slot):
            @pl.when(jnp.logical_and(g >= 1, g <= n_chunks))
            def _():
                s1 = (slot - 1) % _NB
                c_copy(g - 1, s1).wait()
                pltpu.async_copy(qtab.at[ids_v.at[s1, 0]], buf.at[s1],
                                 sg[s1], add=True)
                pltpu.async_copy(ptab.at[ids_v.at[s1, 2]], buf.at[s1],
                                 sg[s1], add=True)

            @pl.when(jnp.logical_and(g >= 2, g <= n_chunks + 1))
            def _():
                s2 = (slot - 2) % _NB
                d = pltpu.make_async_copy(qtab.at[ids_v.at[s2, 0]],
                                          buf.at[s2], sg[s2])
                d.wait()
                d.wait()
                out_copy(g - 2, s2).start()

            @pl.when(g + 2 < n_chunks)
            def _():
                ids_copy(g + 2, (slot + 2) % _NB).start()

            @pl.when(g < n_chunks)
            def _():
                ids_copy(g, slot).wait()

                @pl.when(g >= _NB)
                def _():
                    out_copy(g - _NB, slot).wait()

                c_copy(g, slot).start()

        def body(i, carry):
            for b in range(_NB):
                step_slot(i * _NB + b, b)
            return carry

        n_steps = n_chunks + 2
        lax.fori_loop(0, (n_steps + _NB - 1) // _NB, body, 0)

        for k in range(n_chunks - _NB, n_chunks):
            out_copy(k, k % _NB).wait()

    return sc_kernel


def kernel(question_ids, concept_ids, question_table, concept_table,
           position_table):
    B, S = question_ids.shape
    N = B * S
    RB = 256
    qf = question_ids.reshape(N).astype(jnp.int32)
    cf = concept_ids.reshape(N).astype(jnp.int32)
    pf = jnp.tile(jnp.arange(S, dtype=jnp.int32), B)
    ids_packed = jnp.stack([qf, cf, pf], axis=0).reshape(3, N // RB, RB)
    ids_packed = ids_packed.transpose(1, 0, 2)
    out = _build_sc_kernel(N, RB)(ids_packed, question_table, concept_table,
                                  position_table)
    return out.reshape(B, S, _H)
